# baseline (speedup 1.0000x reference)
import functools

import jax
import jax.numpy as jnp
from jax import lax
from jax.experimental import pallas as pl
from jax.experimental.pallas import tpu as pltpu
from jax.experimental.pallas import tpu_sc as plsc

_CHUNK = 512


@functools.lru_cache(maxsize=None)
def _make_gather(B, V, D, C):
    info = plsc.get_sparse_core_info()
    nw = info.num_cores * info.num_subcores
    b_per_w = B // nw
    n_chunks = b_per_w // C
    mesh = plsc.VectorSubcoreMesh(core_axis_name="c", subcore_axis_name="s")

    @functools.partial(
        pl.kernel,
        out_type=jax.ShapeDtypeStruct((B, D), jnp.float32),
        mesh=mesh,
        scratch_types=[
            pltpu.VMEM((C,), jnp.int32),
            pltpu.VMEM((C, D), jnp.float32),
            pltpu.SemaphoreType.DMA,
        ],
        compiler_params=pltpu.CompilerParams(use_tc_tiling_on_sc=False),
    )
    def gather_kernel(x_hbm, w_hbm, out_hbm, idx_v, rows_v, sem):
        wid = lax.axis_index("s") * info.num_cores + lax.axis_index("c")
        base = wid * b_per_w

        def body(i, carry):
            off = base + i * C
            pltpu.sync_copy(x_hbm.at[pl.ds(off, C)], idx_v)
            pltpu.async_copy(w_hbm.at[idx_v], rows_v, sem).wait()
            pltpu.sync_copy(rows_v, out_hbm.at[pl.ds(off, C)])
            return carry

        lax.fori_loop(0, n_chunks, body, 0)

    return gather_kernel


def kernel(x, weight):
    s0, s1 = x.shape
    B = s0 * s1
    V, D = weight.shape
    flat = x.reshape(B).astype(jnp.int32)
    out = _make_gather(B, V, D, _CHUNK)(flat, weight)
    return out.reshape(s0, s1, D)


# speedup vs baseline: 1.0806x; 1.0806x over previous
import functools

import jax
import jax.numpy as jnp
from jax import lax
from jax.experimental import pallas as pl
from jax.experimental.pallas import tpu as pltpu
from jax.experimental.pallas import tpu_sc as plsc

_CHUNK = 800


@functools.lru_cache(maxsize=None)
def _make_gather(B, V, D, C):
    info = plsc.get_sparse_core_info()
    nw = info.num_cores * info.num_subcores
    b_per_w = B // nw
    n_chunks = b_per_w // C
    n2 = n_chunks // 2
    mesh = plsc.VectorSubcoreMesh(core_axis_name="c", subcore_axis_name="s")

    @functools.partial(
        pl.kernel,
        out_type=jax.ShapeDtypeStruct((B, D), jnp.float32),
        mesh=mesh,
        scratch_types=[
            pltpu.VMEM((C,), jnp.int32),
            pltpu.VMEM((C,), jnp.int32),
            pltpu.VMEM((C, D), jnp.float32),
            pltpu.VMEM((C, D), jnp.float32),
            pltpu.SemaphoreType.DMA,
            pltpu.SemaphoreType.DMA,
            pltpu.SemaphoreType.DMA,
            pltpu.SemaphoreType.DMA,
        ],
        compiler_params=pltpu.CompilerParams(use_tc_tiling_on_sc=False),
    )
    def gather_kernel(x_hbm, w_hbm, out_hbm, idx0, idx1, rows0, rows1,
                      g0, g1, w0, w1):
        wid = lax.axis_index("s") * info.num_cores + lax.axis_index("c")
        base = wid * b_per_w

        def gather0():
            return pltpu.make_async_copy(w_hbm.at[idx0], rows0, g0)

        def gather1():
            return pltpu.make_async_copy(w_hbm.at[idx1], rows1, g1)

        def wb0(off):
            return pltpu.make_async_copy(rows0, out_hbm.at[pl.ds(off, C)], w0)

        def wb1(off):
            return pltpu.make_async_copy(rows1, out_hbm.at[pl.ds(off, C)], w1)

        pltpu.sync_copy(x_hbm.at[pl.ds(base, C)], idx0)
        gather0().start()

        def body(j, carry):
            c0 = base + (2 * j) * C
            c1 = c0 + C
            c2 = c1 + C
            pltpu.sync_copy(x_hbm.at[pl.ds(c1, C)], idx1)

            @pl.when(j > 0)
            def _():
                wb1(c1).wait()

            gather1().start()
            gather0().wait()
            wb0(c0).start()

            @pl.when(j < n2 - 1)
            def _():
                pltpu.sync_copy(x_hbm.at[pl.ds(c2, C)], idx0)
                wb0(c0).wait()
                gather0().start()

            gather1().wait()
            wb1(c1).start()
            return carry

        lax.fori_loop(0, n2, body, 0)
        wb0(base).wait()
        wb1(base).wait()

    return gather_kernel


def kernel(x, weight):
    s0, s1 = x.shape
    B = s0 * s1
    V, D = weight.shape
    flat = x.reshape(B).astype(jnp.int32)
    out = _make_gather(B, V, D, _CHUNK)(flat, weight)
    return out.reshape(s0, s1, D)
